# Initial kernel scaffold; baseline (speedup 1.0000x reference)
import jax
import jax.numpy as jnp
from jax.experimental import pallas as pl


def kernel(w, c, neg, temp, node_emb, ctx_emb, W_comm):
    raise NotImplementedError("write your pallas kernel here")


# speedup vs baseline: 3.5486x; 3.5486x over previous
import functools

import jax
import jax.numpy as jnp
from jax import lax
from jax.experimental import pallas as pl
from jax.experimental.pallas import tpu as pltpu
from jax.experimental.pallas import tpu_sc as plsc

_NC = 2
_NS = 16
_NW = _NC * _NS
_CHUNK = 128


def _log_sigmoid(x):
    return jnp.minimum(x, 0.0) - jnp.log(1.0 + jnp.exp(-jnp.abs(x)))


def _sc_gather(node_emb, ctx_emb, w2d, c2d, n2d, bs, nneg):
    emb = node_emb.shape[1]
    f32 = jnp.float32
    wpw = bs // _NW
    wch = wpw // _CHUNK
    npw = bs * nneg // _NW
    nch = npw // _CHUNK

    mesh = plsc.VectorSubcoreMesh(
        core_axis_name="c", subcore_axis_name="s",
        num_cores=_NC, num_subcores=_NS)

    @functools.partial(
        pl.kernel,
        mesh=mesh,
        out_type=[
            jax.ShapeDtypeStruct((bs, emb), f32),
            jax.ShapeDtypeStruct((bs, emb), f32),
            jax.ShapeDtypeStruct((bs, emb), f32),
            jax.ShapeDtypeStruct((bs * nneg, emb), f32),
        ],
        scratch_types=[
            pltpu.VMEM((wch, _CHUNK), jnp.int32),
            pltpu.VMEM((wch, _CHUNK), jnp.int32),
            pltpu.VMEM((nch, _CHUNK), jnp.int32),
            pltpu.VMEM((_CHUNK, emb), f32),
            pltpu.SemaphoreType.DMA,
        ],
    )
    def gk(node_hbm, ctx_hbm, w_hbm, c_hbm, n_hbm,
           owe, oce, octx, oneg, iw, ic, inn, rows, sem):
        wid = lax.axis_index("s") * _NC + lax.axis_index("c")
        pltpu.sync_copy(w_hbm.at[wid], iw)
        pltpu.sync_copy(c_hbm.at[wid], ic)
        pltpu.sync_copy(n_hbm.at[wid], inn)

        def phase(idx2, nchunks, table, out, base):
            def body(j, carry):
                pltpu.async_copy(table.at[idx2.at[j]], rows, sem).wait()
                pltpu.sync_copy(rows, out.at[pl.ds(base + j * _CHUNK, _CHUNK)])
                return carry
            lax.fori_loop(0, nchunks, body, 0)

        phase(iw, wch, node_hbm, owe, wid * wpw)
        phase(ic, wch, node_hbm, oce, wid * wpw)
        phase(ic, wch, ctx_hbm, octx, wid * wpw)
        phase(inn, nch, ctx_hbm, oneg, wid * npw)

    return gk(node_emb, ctx_emb, w2d, c2d, n2d)


def _tc_math(we, ce, cctx, neg3, w_comm, bs):
    blk = 1024
    nneg, _, emb = neg3.shape
    cat = w_comm.shape[0]
    f32 = jnp.float32

    def body(we_ref, ce_ref, cc_ref, ng_ref, w_ref, sq_ref, pr_ref, acc_ref):
        i = pl.program_id(0)
        w = w_ref[...]
        we_ = we_ref[...]
        dn_t = (((1,), (1,)), ((), ()))
        q = lax.dot_general(we_ * ce_ref[...], w, dn_t,
                            preferred_element_type=f32)
        prior_logits = lax.dot_general(we_, w, dn_t,
                                       preferred_element_type=f32)
        pr_ref[...] = jax.nn.softmax(prior_logits, axis=-1)
        sq_ref[...] = jax.nn.softmax(q, axis=-1)
        r = lax.dot_general(q, w, (((1,), (0,)), ((), ())),
                            preferred_element_type=f32)
        pos = jnp.sum(cc_ref[...] * r, axis=1)
        tot = jnp.sum(_log_sigmoid(pos))
        for n in range(nneg):
            npd = jnp.sum(ng_ref[n] * r, axis=1)
            tot = tot + jnp.sum(_log_sigmoid(-npd))

        @pl.when(i == 0)
        def _():
            acc_ref[0, 0] = 0.0

        acc_ref[0, 0] += tot

    return pl.pallas_call(
        body,
        grid=(bs // blk,),
        in_specs=[
            pl.BlockSpec((blk, emb), lambda i: (i, 0)),
            pl.BlockSpec((blk, emb), lambda i: (i, 0)),
            pl.BlockSpec((blk, emb), lambda i: (i, 0)),
            pl.BlockSpec((nneg, blk, emb), lambda i: (0, i, 0)),
            pl.BlockSpec((cat, emb), lambda i: (0, 0)),
        ],
        out_specs=[
            pl.BlockSpec((blk, cat), lambda i: (i, 0)),
            pl.BlockSpec((blk, cat), lambda i: (i, 0)),
            pl.BlockSpec(memory_space=pltpu.SMEM),
        ],
        out_shape=[
            jax.ShapeDtypeStruct((bs, cat), f32),
            jax.ShapeDtypeStruct((bs, cat), f32),
            jax.ShapeDtypeStruct((1, 1), f32),
        ],
    )(we, ce, cctx, neg3, w_comm)


def kernel(w, c, neg, temp, node_emb, ctx_emb, W_comm):
    bs = w.shape[0]
    nneg = neg.shape[1]
    emb = node_emb.shape[1]
    w = w.astype(jnp.int32)
    c = c.astype(jnp.int32)
    negf = neg.astype(jnp.int32).T.reshape(-1)
    w2d = w.reshape(_NW, -1, _CHUNK)
    c2d = c.reshape(_NW, -1, _CHUNK)
    n2d = negf.reshape(_NW, -1, _CHUNK)
    we, ce, cctx, negrows = _sc_gather(node_emb, ctx_emb, w2d, c2d, n2d,
                                       bs, nneg)
    neg3 = negrows.reshape(nneg, bs, emb)
    sq, prior, acc = _tc_math(we, ce, cctx, neg3, W_comm, bs)
    out = -acc[0, 0] / bs
    return (out, sq, prior)


# speedup vs baseline: 4.0201x; 1.1329x over previous
import functools

import jax
import jax.numpy as jnp
from jax import lax
from jax.experimental import pallas as pl
from jax.experimental.pallas import tpu as pltpu
from jax.experimental.pallas import tpu_sc as plsc

_NC = 2
_NS = 16
_NW = _NC * _NS
_CHUNK = 128
_GRP = 2


def _log_sigmoid(x):
    return jnp.minimum(x, 0.0) - jnp.log(1.0 + jnp.exp(-jnp.abs(x)))


def _sc_gather(node_emb, ctx_emb, nidx3, cidx3, bs, nneg):
    emb = node_emb.shape[1]
    f32 = jnp.float32
    wpw = bs // _NW
    wch = wpw // _CHUNK
    npw = bs * nneg // _NW
    nch = npw // _CHUNK
    grows = _GRP * _CHUNK

    mesh = plsc.VectorSubcoreMesh(
        core_axis_name="c", subcore_axis_name="s",
        num_cores=_NC, num_subcores=_NS)

    @functools.partial(
        pl.kernel,
        mesh=mesh,
        out_type=[
            jax.ShapeDtypeStruct((bs, emb), f32),
            jax.ShapeDtypeStruct((bs, emb), f32),
            jax.ShapeDtypeStruct((bs, emb), f32),
            jax.ShapeDtypeStruct((bs * nneg, emb), f32),
        ],
        scratch_types=[
            pltpu.VMEM((2 * wch, _CHUNK), jnp.int32),
            pltpu.VMEM((wch + nch, _CHUNK), jnp.int32),
            pltpu.VMEM((2 * grows, emb), f32),
            pltpu.SemaphoreType.DMA,
            pltpu.SemaphoreType.DMA,
            pltpu.SemaphoreType.DMA,
            pltpu.SemaphoreType.DMA,
        ],
    )
    def gk(node_hbm, ctx_hbm, ni_hbm, ci_hbm,
           owe, oce, octx, oneg, ino, ict, buf, sg0, sg1, sw0, sw1):
        wid = lax.axis_index("s") * _NC + lax.axis_index("c")
        pltpu.sync_copy(ni_hbm.at[wid], ino)
        pltpu.sync_copy(ci_hbm.at[wid], ict)

        glist = []
        for g in range(wch // _GRP):
            glist.append((node_hbm, ino, g * _GRP, owe,
                          wid * wpw + g * grows))
        for g in range(wch // _GRP):
            glist.append((node_hbm, ino, wch + g * _GRP, oce,
                          wid * wpw + g * grows))
        for g in range(wch // _GRP):
            glist.append((ctx_hbm, ict, g * _GRP, octx,
                          wid * wpw + g * grows))
        for g in range(nch // _GRP):
            glist.append((ctx_hbm, ict, wch + g * _GRP, oneg,
                          wid * npw + g * grows))
        ng = len(glist)
        sg = (sg0, sg1)
        sw = (sw0, sw1)

        def fire(g):
            tbl, ibuf, ibase, _, _ = glist[g]
            s = g % 2
            return [
                pltpu.async_copy(
                    tbl.at[ibuf.at[ibase + j]],
                    buf.at[pl.ds(s * grows + j * _CHUNK, _CHUNK)],
                    sg[s])
                for j in range(_GRP)
            ]

        gdesc = {0: fire(0)}
        wdesc = [None, None]
        for g in range(ng):
            s = g % 2
            for d in gdesc.pop(g):
                d.wait()
            _, _, _, out, obase = glist[g]
            wdesc[s] = pltpu.async_copy(
                buf.at[pl.ds(s * grows, grows)],
                out.at[pl.ds(obase, grows)], sw[s])
            if g + 1 < ng:
                s2 = (g + 1) % 2
                if wdesc[s2] is not None:
                    wdesc[s2].wait()
                gdesc[g + 1] = fire(g + 1)
        for d in wdesc:
            if d is not None:
                d.wait()

    return gk(node_emb, ctx_emb, nidx3, cidx3)


def _tc_math(we, ce, cctx, neg3, w_comm, bs):
    blk = 1024
    nneg, _, emb = neg3.shape
    cat = w_comm.shape[0]
    f32 = jnp.float32

    def body(we_ref, ce_ref, cc_ref, ng_ref, w_ref, sq_ref, pr_ref, acc_ref):
        i = pl.program_id(0)
        w = w_ref[...]
        we_ = we_ref[...]
        dn_t = (((1,), (1,)), ((), ()))
        q = lax.dot_general(we_ * ce_ref[...], w, dn_t,
                            preferred_element_type=f32)
        prior_logits = lax.dot_general(we_, w, dn_t,
                                       preferred_element_type=f32)
        pr_ref[...] = jax.nn.softmax(prior_logits, axis=-1)
        sq_ref[...] = jax.nn.softmax(q, axis=-1)
        r = lax.dot_general(q, w, (((1,), (0,)), ((), ())),
                            preferred_element_type=f32)
        pos = jnp.sum(cc_ref[...] * r, axis=1)
        tot = jnp.sum(_log_sigmoid(pos))
        for n in range(nneg):
            npd = jnp.sum(ng_ref[n] * r, axis=1)
            tot = tot + jnp.sum(_log_sigmoid(-npd))

        @pl.when(i == 0)
        def _():
            acc_ref[0, 0] = 0.0

        acc_ref[0, 0] += tot

    return pl.pallas_call(
        body,
        grid=(bs // blk,),
        in_specs=[
            pl.BlockSpec((blk, emb), lambda i: (i, 0)),
            pl.BlockSpec((blk, emb), lambda i: (i, 0)),
            pl.BlockSpec((blk, emb), lambda i: (i, 0)),
            pl.BlockSpec((nneg, blk, emb), lambda i: (0, i, 0)),
            pl.BlockSpec((cat, emb), lambda i: (0, 0)),
        ],
        out_specs=[
            pl.BlockSpec((blk, cat), lambda i: (i, 0)),
            pl.BlockSpec((blk, cat), lambda i: (i, 0)),
            pl.BlockSpec(memory_space=pltpu.SMEM),
        ],
        out_shape=[
            jax.ShapeDtypeStruct((bs, cat), f32),
            jax.ShapeDtypeStruct((bs, cat), f32),
            jax.ShapeDtypeStruct((1, 1), f32),
        ],
    )(we, ce, cctx, neg3, w_comm)


def kernel(w, c, neg, temp, node_emb, ctx_emb, W_comm):
    bs = w.shape[0]
    nneg = neg.shape[1]
    emb = node_emb.shape[1]
    w = w.astype(jnp.int32)
    c = c.astype(jnp.int32)
    negf = neg.astype(jnp.int32).T.reshape(-1)
    nidx3 = jnp.concatenate(
        [w.reshape(_NW, -1), c.reshape(_NW, -1)], axis=1
    ).reshape(_NW, -1, _CHUNK)
    cidx3 = jnp.concatenate(
        [c.reshape(_NW, -1), negf.reshape(_NW, -1)], axis=1
    ).reshape(_NW, -1, _CHUNK)
    we, ce, cctx, negrows = _sc_gather(node_emb, ctx_emb, nidx3, cidx3,
                                       bs, nneg)
    neg3 = negrows.reshape(nneg, bs, emb)
    sq, prior, acc = _tc_math(we, ce, cctx, neg3, W_comm, bs)
    out = -acc[0, 0] / bs
    return (out, sq, prior)


# speedup vs baseline: 4.0433x; 1.0058x over previous
import functools

import jax
import jax.numpy as jnp
from jax import lax
from jax.experimental import pallas as pl
from jax.experimental.pallas import tpu as pltpu
from jax.experimental.pallas import tpu_sc as plsc

_NC = 2
_NS = 16
_NW = _NC * _NS
_CHUNK = 128
_GRP = 2
_NH = 2


def _log_sigmoid(x):
    return jnp.minimum(x, 0.0) - jnp.log(1.0 + jnp.exp(-jnp.abs(x)))


def _sc_gather(node_emb, ctx_emb, nidx3, cidx3, bs, nneg):
    emb = node_emb.shape[1]
    f32 = jnp.float32
    wpw = bs // _NW
    wch = wpw // _CHUNK
    npw = bs * nneg // _NW
    nch = npw // _CHUNK
    grows = _GRP * _CHUNK

    mesh = plsc.VectorSubcoreMesh(
        core_axis_name="c", subcore_axis_name="s",
        num_cores=_NC, num_subcores=_NS)

    @functools.partial(
        pl.kernel,
        mesh=mesh,
        out_type=[
            jax.ShapeDtypeStruct((bs, emb), f32),
            jax.ShapeDtypeStruct((bs, emb), f32),
            jax.ShapeDtypeStruct((bs, emb), f32),
            jax.ShapeDtypeStruct((bs * nneg, emb), f32),
        ],
        scratch_types=[
            pltpu.VMEM((2 * wch, _CHUNK), jnp.int32),
            pltpu.VMEM((wch + nch, _CHUNK), jnp.int32),
            pltpu.VMEM((2 * grows, emb), f32),
            pltpu.SemaphoreType.DMA,
            pltpu.SemaphoreType.DMA,
            pltpu.SemaphoreType.DMA,
            pltpu.SemaphoreType.DMA,
        ],
    )
    def gk(node_hbm, ctx_hbm, ni_hbm, ci_hbm,
           owe, oce, octx, oneg, ino, ict, buf, sg0, sg1, sw0, sw1):
        wid = lax.axis_index("s") * _NC + lax.axis_index("c")
        pltpu.sync_copy(ni_hbm.at[wid], ino)
        pltpu.sync_copy(ci_hbm.at[wid], ict)

        glist = []
        for g in range(wch // _GRP):
            glist.append((node_hbm, ino, g * _GRP, owe,
                          wid * wpw + g * grows))
        for g in range(wch // _GRP):
            glist.append((node_hbm, ino, wch + g * _GRP, oce,
                          wid * wpw + g * grows))
        for g in range(wch // _GRP):
            glist.append((ctx_hbm, ict, g * _GRP, octx,
                          wid * wpw + g * grows))
        for g in range(nch // _GRP):
            glist.append((ctx_hbm, ict, wch + g * _GRP, oneg,
                          wid * npw + g * grows))
        ng = len(glist)
        sg = (sg0, sg1)
        sw = (sw0, sw1)

        def fire(g):
            tbl, ibuf, ibase, _, _ = glist[g]
            s = g % 2
            return [
                pltpu.async_copy(
                    tbl.at[ibuf.at[ibase + j]],
                    buf.at[pl.ds(s * grows + j * _CHUNK, _CHUNK)],
                    sg[s])
                for j in range(_GRP)
            ]

        gdesc = {0: fire(0)}
        wdesc = [None, None]
        for g in range(ng):
            s = g % 2
            for d in gdesc.pop(g):
                d.wait()
            _, _, _, out, obase = glist[g]
            wdesc[s] = pltpu.async_copy(
                buf.at[pl.ds(s * grows, grows)],
                out.at[pl.ds(obase, grows)], sw[s])
            if g + 1 < ng:
                s2 = (g + 1) % 2
                if wdesc[s2] is not None:
                    wdesc[s2].wait()
                gdesc[g + 1] = fire(g + 1)
        for d in wdesc:
            if d is not None:
                d.wait()

    return gk(node_emb, ctx_emb, nidx3, cidx3)


def _tc_math(we, ce, cctx, neg3, w_comm, bs, h, sq_prev, pr_prev):
    blk = 1024
    nneg, bsh, emb = neg3.shape
    cat = w_comm.shape[0]
    f32 = jnp.float32
    nb = bsh // blk
    ob = h * nb

    def body(we_ref, ce_ref, cc_ref, ng_ref, w_ref, _sq_in, _pr_in,
             sq_ref, pr_ref, acc_ref):
        i = pl.program_id(0)
        w = w_ref[...]
        we_ = we_ref[...]
        dn_t = (((1,), (1,)), ((), ()))
        q = lax.dot_general(we_ * ce_ref[...], w, dn_t,
                            preferred_element_type=f32)
        prior_logits = lax.dot_general(we_, w, dn_t,
                                       preferred_element_type=f32)
        pr_ref[...] = jax.nn.softmax(prior_logits, axis=-1)
        sq_ref[...] = jax.nn.softmax(q, axis=-1)
        r = lax.dot_general(q, w, (((1,), (0,)), ((), ())),
                            preferred_element_type=f32)
        pos = jnp.sum(cc_ref[...] * r, axis=1)
        tot = jnp.sum(_log_sigmoid(pos))
        for n in range(nneg):
            npd = jnp.sum(ng_ref[n] * r, axis=1)
            tot = tot + jnp.sum(_log_sigmoid(-npd))

        @pl.when(i == 0)
        def _():
            acc_ref[0, 0] = 0.0

        acc_ref[0, 0] += tot

    return pl.pallas_call(
        body,
        grid=(nb,),
        in_specs=[
            pl.BlockSpec((blk, emb), lambda i: (i, 0)),
            pl.BlockSpec((blk, emb), lambda i: (i, 0)),
            pl.BlockSpec((blk, emb), lambda i: (i, 0)),
            pl.BlockSpec((nneg, blk, emb), lambda i: (0, i, 0)),
            pl.BlockSpec((cat, emb), lambda i: (0, 0)),
            pl.BlockSpec(memory_space=pl.ANY),
            pl.BlockSpec(memory_space=pl.ANY),
        ],
        out_specs=[
            pl.BlockSpec((blk, cat), lambda i: (i + ob, 0)),
            pl.BlockSpec((blk, cat), lambda i: (i + ob, 0)),
            pl.BlockSpec(memory_space=pltpu.SMEM),
        ],
        out_shape=[
            jax.ShapeDtypeStruct((bs, cat), f32),
            jax.ShapeDtypeStruct((bs, cat), f32),
            jax.ShapeDtypeStruct((1, 1), f32),
        ],
        input_output_aliases={5: 0, 6: 1},
    )(we, ce, cctx, neg3, w_comm, sq_prev, pr_prev)


def kernel(w, c, neg, temp, node_emb, ctx_emb, W_comm):
    bs = w.shape[0]
    nneg = neg.shape[1]
    emb = node_emb.shape[1]
    cat = W_comm.shape[0]
    f32 = jnp.float32
    w = w.astype(jnp.int32)
    c = c.astype(jnp.int32)
    neg = neg.astype(jnp.int32)
    bsh = bs // _NH

    halves = []
    for h in range(_NH):
        wh = lax.dynamic_slice_in_dim(w, h * bsh, bsh)
        ch = lax.dynamic_slice_in_dim(c, h * bsh, bsh)
        negf = lax.dynamic_slice_in_dim(neg, h * bsh, bsh).T.reshape(-1)
        nidx3 = jnp.concatenate(
            [wh.reshape(_NW, -1), ch.reshape(_NW, -1)], axis=1
        ).reshape(_NW, -1, _CHUNK)
        cidx3 = jnp.concatenate(
            [ch.reshape(_NW, -1), negf.reshape(_NW, -1)], axis=1
        ).reshape(_NW, -1, _CHUNK)
        halves.append((nidx3, cidx3))

    sq = jnp.zeros((bs, cat), f32)
    prior = jnp.zeros((bs, cat), f32)
    tot = jnp.float32(0.0)
    for h in range(_NH):
        nidx3, cidx3 = halves[h]
        we, ce, cctx, negrows = _sc_gather(node_emb, ctx_emb, nidx3, cidx3,
                                           bsh, nneg)
        neg3 = negrows.reshape(nneg, bsh, emb)
        sq, prior, acc = _tc_math(we, ce, cctx, neg3, W_comm, bs, h,
                                  sq, prior)
        tot = tot + acc[0, 0]
    out = -tot / bs
    return (out, sq, prior)


# speedup vs baseline: 4.0899x; 1.0115x over previous
import functools

import jax
import jax.numpy as jnp
from jax import lax
from jax.experimental import pallas as pl
from jax.experimental.pallas import tpu as pltpu
from jax.experimental.pallas import tpu_sc as plsc

_NC = 2
_NS = 16
_NW = _NC * _NS
_CHUNK = 128
_GRP = 2
_NBUF = 3
_NH = 2


def _log_sigmoid(x):
    return jnp.minimum(x, 0.0) - jnp.log(1.0 + jnp.exp(-jnp.abs(x)))


def _sc_gather(node_emb, ctx_emb, nidx3, cidx3, bs, nneg):
    emb = node_emb.shape[1]
    f32 = jnp.float32
    wpw = bs // _NW
    wch = wpw // _CHUNK
    npw = bs * nneg // _NW
    nch = npw // _CHUNK
    grows = _GRP * _CHUNK

    mesh = plsc.VectorSubcoreMesh(
        core_axis_name="c", subcore_axis_name="s",
        num_cores=_NC, num_subcores=_NS)

    @functools.partial(
        pl.kernel,
        mesh=mesh,
        out_type=[
            jax.ShapeDtypeStruct((bs, emb), f32),
            jax.ShapeDtypeStruct((bs, emb), f32),
            jax.ShapeDtypeStruct((bs, emb), f32),
            jax.ShapeDtypeStruct((bs * nneg, emb), f32),
        ],
        scratch_types=[
            pltpu.VMEM((2 * wch, _CHUNK), jnp.int32),
            pltpu.VMEM((wch + nch, _CHUNK), jnp.int32),
            pltpu.VMEM((_NBUF * grows, emb), f32),
            pltpu.SemaphoreType.DMA,
            pltpu.SemaphoreType.DMA,
            pltpu.SemaphoreType.DMA,
            pltpu.SemaphoreType.DMA,
            pltpu.SemaphoreType.DMA,
            pltpu.SemaphoreType.DMA,
        ],
    )
    def gk(node_hbm, ctx_hbm, ni_hbm, ci_hbm,
           owe, oce, octx, oneg, ino, ict, buf,
           sg0, sg1, sg2, sw0, sw1, sw2):
        wid = lax.axis_index("s") * _NC + lax.axis_index("c")
        pltpu.sync_copy(ni_hbm.at[wid], ino)
        pltpu.sync_copy(ci_hbm.at[wid], ict)

        glist = []
        for g in range(wch // _GRP):
            glist.append((node_hbm, ino, g * _GRP, owe,
                          wid * wpw + g * grows))
        for g in range(wch // _GRP):
            glist.append((node_hbm, ino, wch + g * _GRP, oce,
                          wid * wpw + g * grows))
        for g in range(wch // _GRP):
            glist.append((ctx_hbm, ict, g * _GRP, octx,
                          wid * wpw + g * grows))
        for g in range(nch // _GRP):
            glist.append((ctx_hbm, ict, wch + g * _GRP, oneg,
                          wid * npw + g * grows))
        ng = len(glist)
        sg = (sg0, sg1, sg2)
        sw = (sw0, sw1, sw2)

        def fire(g):
            tbl, ibuf, ibase, _, _ = glist[g]
            s = g % _NBUF
            return [
                pltpu.async_copy(
                    tbl.at[ibuf.at[ibase + j]],
                    buf.at[pl.ds(s * grows + j * _CHUNK, _CHUNK)],
                    sg[s])
                for j in range(_GRP)
            ]

        gdesc = {}
        wdesc = {}
        for k in range(min(_NBUF - 1, ng)):
            gdesc[k] = fire(k)
        for g in range(ng):
            s = g % _NBUF
            for d in gdesc.pop(g):
                d.wait()
            _, _, _, out, obase = glist[g]
            wdesc[g] = pltpu.async_copy(
                buf.at[pl.ds(s * grows, grows)],
                out.at[pl.ds(obase, grows)], sw[s])
            nxt = g + _NBUF - 1
            if nxt < ng:
                prev = nxt - _NBUF
                if prev in wdesc:
                    wdesc.pop(prev).wait()
                gdesc[nxt] = fire(nxt)
        for g in sorted(wdesc):
            wdesc.pop(g).wait()

    return gk(node_emb, ctx_emb, nidx3, cidx3)


def _tc_math(we, ce, cctx, neg3, w_comm, bs, h, sq_prev, pr_prev):
    blk = 1024
    nneg, bsh, emb = neg3.shape
    cat = w_comm.shape[0]
    f32 = jnp.float32
    nb = bsh // blk
    ob = h * nb

    def body(we_ref, ce_ref, cc_ref, ng_ref, w_ref, _sq_in, _pr_in,
             sq_ref, pr_ref, acc_ref):
        i = pl.program_id(0)
        w = w_ref[...]
        we_ = we_ref[...]
        dn_t = (((1,), (1,)), ((), ()))
        q = lax.dot_general(we_ * ce_ref[...], w, dn_t,
                            preferred_element_type=f32)
        prior_logits = lax.dot_general(we_, w, dn_t,
                                       preferred_element_type=f32)
        pr_ref[...] = jax.nn.softmax(prior_logits, axis=-1)
        sq_ref[...] = jax.nn.softmax(q, axis=-1)
        r = lax.dot_general(q, w, (((1,), (0,)), ((), ())),
                            preferred_element_type=f32)
        pos = jnp.sum(cc_ref[...] * r, axis=1)
        tot = jnp.sum(_log_sigmoid(pos))
        for n in range(nneg):
            npd = jnp.sum(ng_ref[n] * r, axis=1)
            tot = tot + jnp.sum(_log_sigmoid(-npd))

        @pl.when(i == 0)
        def _():
            acc_ref[0, 0] = 0.0

        acc_ref[0, 0] += tot

    return pl.pallas_call(
        body,
        grid=(nb,),
        in_specs=[
            pl.BlockSpec((blk, emb), lambda i: (i, 0)),
            pl.BlockSpec((blk, emb), lambda i: (i, 0)),
            pl.BlockSpec((blk, emb), lambda i: (i, 0)),
            pl.BlockSpec((nneg, blk, emb), lambda i: (0, i, 0)),
            pl.BlockSpec((cat, emb), lambda i: (0, 0)),
            pl.BlockSpec(memory_space=pl.ANY),
            pl.BlockSpec(memory_space=pl.ANY),
        ],
        out_specs=[
            pl.BlockSpec((blk, cat), lambda i: (i + ob, 0)),
            pl.BlockSpec((blk, cat), lambda i: (i + ob, 0)),
            pl.BlockSpec(memory_space=pltpu.SMEM),
        ],
        out_shape=[
            jax.ShapeDtypeStruct((bs, cat), f32),
            jax.ShapeDtypeStruct((bs, cat), f32),
            jax.ShapeDtypeStruct((1, 1), f32),
        ],
        input_output_aliases={5: 0, 6: 1},
    )(we, ce, cctx, neg3, w_comm, sq_prev, pr_prev)


def kernel(w, c, neg, temp, node_emb, ctx_emb, W_comm):
    bs = w.shape[0]
    nneg = neg.shape[1]
    emb = node_emb.shape[1]
    cat = W_comm.shape[0]
    f32 = jnp.float32
    w = w.astype(jnp.int32)
    c = c.astype(jnp.int32)
    neg = neg.astype(jnp.int32)
    bsh = bs // _NH

    halves = []
    for h in range(_NH):
        wh = lax.dynamic_slice_in_dim(w, h * bsh, bsh)
        ch = lax.dynamic_slice_in_dim(c, h * bsh, bsh)
        negf = lax.dynamic_slice_in_dim(neg, h * bsh, bsh).T.reshape(-1)
        nidx3 = jnp.concatenate(
            [wh.reshape(_NW, -1), ch.reshape(_NW, -1)], axis=1
        ).reshape(_NW, -1, _CHUNK)
        cidx3 = jnp.concatenate(
            [ch.reshape(_NW, -1), negf.reshape(_NW, -1)], axis=1
        ).reshape(_NW, -1, _CHUNK)
        halves.append((nidx3, cidx3))

    sq = jnp.zeros((bs, cat), f32)
    prior = jnp.zeros((bs, cat), f32)
    tot = jnp.float32(0.0)
    for h in range(_NH):
        nidx3, cidx3 = halves[h]
        we, ce, cctx, negrows = _sc_gather(node_emb, ctx_emb, nidx3, cidx3,
                                           bsh, nneg)
        neg3 = negrows.reshape(nneg, bsh, emb)
        sq, prior, acc = _tc_math(we, ce, cctx, neg3, W_comm, bs, h,
                                  sq, prior)
        tot = tot + acc[0, 0]
    out = -tot / bs
    return (out, sq, prior)


# speedup vs baseline: 4.1577x; 1.0166x over previous
import functools

import jax
import jax.numpy as jnp
from jax import lax
from jax.experimental import pallas as pl
from jax.experimental.pallas import tpu as pltpu
from jax.experimental.pallas import tpu_sc as plsc

_NC = 2
_NS = 16
_NW = _NC * _NS
_CHUNK = 128
_GRP = 1
_NBUF = 6
_NH = 2


def _log_sigmoid(x):
    return jnp.minimum(x, 0.0) - jnp.log(1.0 + jnp.exp(-jnp.abs(x)))


def _sc_gather(node_emb, ctx_emb, nidx3, cidx3, bs, nneg):
    emb = node_emb.shape[1]
    f32 = jnp.float32
    wpw = bs // _NW
    wch = wpw // _CHUNK
    npw = bs * nneg // _NW
    nch = npw // _CHUNK
    grows = _GRP * _CHUNK

    mesh = plsc.VectorSubcoreMesh(
        core_axis_name="c", subcore_axis_name="s",
        num_cores=_NC, num_subcores=_NS)

    @functools.partial(
        pl.kernel,
        mesh=mesh,
        out_type=[
            jax.ShapeDtypeStruct((bs, emb), f32),
            jax.ShapeDtypeStruct((bs, emb), f32),
            jax.ShapeDtypeStruct((bs, emb), f32),
            jax.ShapeDtypeStruct((bs * nneg, emb), f32),
        ],
        scratch_types=[
            pltpu.VMEM((2 * wch, _CHUNK), jnp.int32),
            pltpu.VMEM((wch + nch, _CHUNK), jnp.int32),
            pltpu.VMEM((_NBUF * grows, emb), f32),
        ] + [pltpu.SemaphoreType.DMA] * (2 * _NBUF),
    )
    def gk(node_hbm, ctx_hbm, ni_hbm, ci_hbm,
           owe, oce, octx, oneg, ino, ict, buf, *sems):
        wid = lax.axis_index("s") * _NC + lax.axis_index("c")
        pltpu.sync_copy(ni_hbm.at[wid], ino)
        pltpu.sync_copy(ci_hbm.at[wid], ict)

        glist = []
        for g in range(wch // _GRP):
            glist.append((node_hbm, ino, g * _GRP, owe,
                          wid * wpw + g * grows))
        for g in range(wch // _GRP):
            glist.append((node_hbm, ino, wch + g * _GRP, oce,
                          wid * wpw + g * grows))
        for g in range(wch // _GRP):
            glist.append((ctx_hbm, ict, g * _GRP, octx,
                          wid * wpw + g * grows))
        for g in range(nch // _GRP):
            glist.append((ctx_hbm, ict, wch + g * _GRP, oneg,
                          wid * npw + g * grows))
        ng = len(glist)
        sg = sems[:_NBUF]
        sw = sems[_NBUF:]

        def fire(g):
            tbl, ibuf, ibase, _, _ = glist[g]
            s = g % _NBUF
            return [
                pltpu.async_copy(
                    tbl.at[ibuf.at[ibase + j]],
                    buf.at[pl.ds(s * grows + j * _CHUNK, _CHUNK)],
                    sg[s])
                for j in range(_GRP)
            ]

        gdesc = {}
        wdesc = {}
        for k in range(min(_NBUF - 1, ng)):
            gdesc[k] = fire(k)
        for g in range(ng):
            s = g % _NBUF
            for d in gdesc.pop(g):
                d.wait()
            _, _, _, out, obase = glist[g]
            wdesc[g] = pltpu.async_copy(
                buf.at[pl.ds(s * grows, grows)],
                out.at[pl.ds(obase, grows)], sw[s])
            nxt = g + _NBUF - 1
            if nxt < ng:
                prev = nxt - _NBUF
                if prev in wdesc:
                    wdesc.pop(prev).wait()
                gdesc[nxt] = fire(nxt)
        for g in sorted(wdesc):
            wdesc.pop(g).wait()

    return gk(node_emb, ctx_emb, nidx3, cidx3)


def _tc_math(we, ce, cctx, neg3, w_comm, bs, h, sq_prev, pr_prev):
    blk = 1024
    nneg, bsh, emb = neg3.shape
    cat = w_comm.shape[0]
    f32 = jnp.float32
    nb = bsh // blk
    ob = h * nb

    def body(we_ref, ce_ref, cc_ref, ng_ref, w_ref, _sq_in, _pr_in,
             sq_ref, pr_ref, acc_ref):
        i = pl.program_id(0)
        w = w_ref[...]
        we_ = we_ref[...]
        dn_t = (((1,), (1,)), ((), ()))
        q = lax.dot_general(we_ * ce_ref[...], w, dn_t,
                            preferred_element_type=f32)
        prior_logits = lax.dot_general(we_, w, dn_t,
                                       preferred_element_type=f32)
        pr_ref[...] = jax.nn.softmax(prior_logits, axis=-1)
        sq_ref[...] = jax.nn.softmax(q, axis=-1)
        r = lax.dot_general(q, w, (((1,), (0,)), ((), ())),
                            preferred_element_type=f32)
        pos = jnp.sum(cc_ref[...] * r, axis=1)
        tot = jnp.sum(_log_sigmoid(pos))
        for n in range(nneg):
            npd = jnp.sum(ng_ref[n] * r, axis=1)
            tot = tot + jnp.sum(_log_sigmoid(-npd))

        @pl.when(i == 0)
        def _():
            acc_ref[0, 0] = 0.0

        acc_ref[0, 0] += tot

    return pl.pallas_call(
        body,
        grid=(nb,),
        in_specs=[
            pl.BlockSpec((blk, emb), lambda i: (i, 0)),
            pl.BlockSpec((blk, emb), lambda i: (i, 0)),
            pl.BlockSpec((blk, emb), lambda i: (i, 0)),
            pl.BlockSpec((nneg, blk, emb), lambda i: (0, i, 0)),
            pl.BlockSpec((cat, emb), lambda i: (0, 0)),
            pl.BlockSpec(memory_space=pl.ANY),
            pl.BlockSpec(memory_space=pl.ANY),
        ],
        out_specs=[
            pl.BlockSpec((blk, cat), lambda i: (i + ob, 0)),
            pl.BlockSpec((blk, cat), lambda i: (i + ob, 0)),
            pl.BlockSpec(memory_space=pltpu.SMEM),
        ],
        out_shape=[
            jax.ShapeDtypeStruct((bs, cat), f32),
            jax.ShapeDtypeStruct((bs, cat), f32),
            jax.ShapeDtypeStruct((1, 1), f32),
        ],
        input_output_aliases={5: 0, 6: 1},
    )(we, ce, cctx, neg3, w_comm, sq_prev, pr_prev)


def kernel(w, c, neg, temp, node_emb, ctx_emb, W_comm):
    bs = w.shape[0]
    nneg = neg.shape[1]
    emb = node_emb.shape[1]
    cat = W_comm.shape[0]
    f32 = jnp.float32
    w = w.astype(jnp.int32)
    c = c.astype(jnp.int32)
    neg = neg.astype(jnp.int32)
    bsh = bs // _NH

    halves = []
    for h in range(_NH):
        wh = lax.dynamic_slice_in_dim(w, h * bsh, bsh)
        ch = lax.dynamic_slice_in_dim(c, h * bsh, bsh)
        negf = lax.dynamic_slice_in_dim(neg, h * bsh, bsh).T.reshape(-1)
        nidx3 = jnp.concatenate(
            [wh.reshape(_NW, -1), ch.reshape(_NW, -1)], axis=1
        ).reshape(_NW, -1, _CHUNK)
        cidx3 = jnp.concatenate(
            [ch.reshape(_NW, -1), negf.reshape(_NW, -1)], axis=1
        ).reshape(_NW, -1, _CHUNK)
        halves.append((nidx3, cidx3))

    sq = jnp.zeros((bs, cat), f32)
    prior = jnp.zeros((bs, cat), f32)
    tot = jnp.float32(0.0)
    for h in range(_NH):
        nidx3, cidx3 = halves[h]
        we, ce, cctx, negrows = _sc_gather(node_emb, ctx_emb, nidx3, cidx3,
                                           bsh, nneg)
        neg3 = negrows.reshape(nneg, bsh, emb)
        sq, prior, acc = _tc_math(we, ce, cctx, neg3, W_comm, bs, h,
                                  sq, prior)
        tot = tot + acc[0, 0]
    out = -tot / bs
    return (out, sq, prior)


# speedup vs baseline: 4.2280x; 1.0169x over previous
import functools

import jax
import jax.numpy as jnp
from jax import lax
from jax.experimental import pallas as pl
from jax.experimental.pallas import tpu as pltpu
from jax.experimental.pallas import tpu_sc as plsc

_NC = 2
_NS = 16
_NW = _NC * _NS
_CHUNK = 128
_GRP = 1
_NBUF = 6
_NH = 2


def _log_sigmoid(x):
    return jnp.minimum(x, 0.0) - jnp.log(1.0 + jnp.exp(-jnp.abs(x)))


def _sc_gather(node_emb, ctx_emb, nidx3, cidx3, bs, nneg):
    emb = node_emb.shape[1]
    f32 = jnp.float32
    wpw = bs // _NW
    wch = wpw // _CHUNK
    npw = bs * nneg // _NW
    nch = npw // _CHUNK
    grows = _GRP * _CHUNK

    mesh = plsc.VectorSubcoreMesh(
        core_axis_name="c", subcore_axis_name="s",
        num_cores=_NC, num_subcores=_NS)

    @functools.partial(
        pl.kernel,
        mesh=mesh,
        out_type=[
            jax.ShapeDtypeStruct((bs, emb), f32),
            jax.ShapeDtypeStruct((bs, emb), f32),
            jax.ShapeDtypeStruct((bs, emb), f32),
            jax.ShapeDtypeStruct((bs * nneg, emb), f32),
        ],
        scratch_types=[
            pltpu.VMEM((2 * wch, _CHUNK), jnp.int32),
            pltpu.VMEM((wch + nch, _CHUNK), jnp.int32),
            pltpu.VMEM((_NBUF * grows, emb), f32),
        ] + [pltpu.SemaphoreType.DMA] * (2 * _NBUF),
    )
    def gk(node_hbm, ctx_hbm, ni_hbm, ci_hbm,
           owe, oce, octx, oneg, ino, ict, buf, *sems):
        wid = lax.axis_index("s") * _NC + lax.axis_index("c")
        pltpu.sync_copy(ni_hbm.at[wid], ino)
        pltpu.sync_copy(ci_hbm.at[wid], ict)

        glist = []
        for g in range(wch // _GRP):
            glist.append((node_hbm, ino, g * _GRP, owe,
                          wid * wpw + g * grows))
        for g in range(wch // _GRP):
            glist.append((node_hbm, ino, wch + g * _GRP, oce,
                          wid * wpw + g * grows))
        for g in range(wch // _GRP):
            glist.append((ctx_hbm, ict, g * _GRP, octx,
                          wid * wpw + g * grows))
        for g in range(nch // _GRP):
            glist.append((ctx_hbm, ict, wch + g * _GRP, oneg,
                          wid * npw + g * grows))
        ng = len(glist)
        sg = sems[:_NBUF]
        sw = sems[_NBUF:]

        def fire(g):
            tbl, ibuf, ibase, _, _ = glist[g]
            s = g % _NBUF
            return [
                pltpu.async_copy(
                    tbl.at[ibuf.at[ibase + j]],
                    buf.at[pl.ds(s * grows + j * _CHUNK, _CHUNK)],
                    sg[s])
                for j in range(_GRP)
            ]

        gdesc = {}
        wdesc = {}
        for k in range(min(_NBUF - 1, ng)):
            gdesc[k] = fire(k)
        for g in range(ng):
            s = g % _NBUF
            for d in gdesc.pop(g):
                d.wait()
            _, _, _, out, obase = glist[g]
            wdesc[g] = pltpu.async_copy(
                buf.at[pl.ds(s * grows, grows)],
                out.at[pl.ds(obase, grows)], sw[s])
            nxt = g + _NBUF - 1
            if nxt < ng:
                prev = nxt - _NBUF
                if prev in wdesc:
                    wdesc.pop(prev).wait()
                gdesc[nxt] = fire(nxt)
        for g in sorted(wdesc):
            wdesc.pop(g).wait()

    return gk(node_emb, ctx_emb, nidx3, cidx3)


def _tc_math(we, ce, cctx, neg3, w_comm, bs, row0, sq_prev, pr_prev):
    blk = 1024
    nneg, bsh, emb = neg3.shape
    cat = w_comm.shape[0]
    f32 = jnp.float32
    nb = bsh // blk
    ob = row0 // blk
    prev = [] if sq_prev is None else [sq_prev, pr_prev]

    def body(we_ref, ce_ref, cc_ref, ng_ref, w_ref, *rest):
        sq_ref, pr_ref, acc_ref = rest[-3:]
        i = pl.program_id(0)
        w = w_ref[...]
        we_ = we_ref[...]
        dn_t = (((1,), (1,)), ((), ()))
        q = lax.dot_general(we_ * ce_ref[...], w, dn_t,
                            preferred_element_type=f32)
        prior_logits = lax.dot_general(we_, w, dn_t,
                                       preferred_element_type=f32)
        pr_ref[...] = jax.nn.softmax(prior_logits, axis=-1)
        sq_ref[...] = jax.nn.softmax(q, axis=-1)
        r = lax.dot_general(q, w, (((1,), (0,)), ((), ())),
                            preferred_element_type=f32)
        pos = jnp.sum(cc_ref[...] * r, axis=1)
        tot = jnp.sum(_log_sigmoid(pos))
        for n in range(nneg):
            npd = jnp.sum(ng_ref[n] * r, axis=1)
            tot = tot + jnp.sum(_log_sigmoid(-npd))

        @pl.when(i == 0)
        def _():
            acc_ref[0, 0] = 0.0

        acc_ref[0, 0] += tot

    return pl.pallas_call(
        body,
        grid=(nb,),
        in_specs=[
            pl.BlockSpec((blk, emb), lambda i: (i, 0)),
            pl.BlockSpec((blk, emb), lambda i: (i, 0)),
            pl.BlockSpec((blk, emb), lambda i: (i, 0)),
            pl.BlockSpec((nneg, blk, emb), lambda i: (0, i, 0)),
            pl.BlockSpec((cat, emb), lambda i: (0, 0)),
        ] + [pl.BlockSpec(memory_space=pl.ANY)] * len(prev),
        out_specs=[
            pl.BlockSpec((blk, cat), lambda i: (i + ob, 0)),
            pl.BlockSpec((blk, cat), lambda i: (i + ob, 0)),
            pl.BlockSpec(memory_space=pltpu.SMEM),
        ],
        out_shape=[
            jax.ShapeDtypeStruct((bs, cat), f32),
            jax.ShapeDtypeStruct((bs, cat), f32),
            jax.ShapeDtypeStruct((1, 1), f32),
        ],
        input_output_aliases={5: 0, 6: 1} if prev else {},
    )(we, ce, cctx, neg3, w_comm, *prev)


def kernel(w, c, neg, temp, node_emb, ctx_emb, W_comm):
    bs = w.shape[0]
    nneg = neg.shape[1]
    emb = node_emb.shape[1]
    cat = W_comm.shape[0]
    f32 = jnp.float32
    w = w.astype(jnp.int32)
    c = c.astype(jnp.int32)
    neg = neg.astype(jnp.int32)
    sizes = (bs * 3 // 4, bs // 4)
    offs = (0, bs * 3 // 4)

    chunks = []
    for o, n in zip(offs, sizes):
        wh = lax.dynamic_slice_in_dim(w, o, n)
        ch = lax.dynamic_slice_in_dim(c, o, n)
        negf = lax.dynamic_slice_in_dim(neg, o, n).T.reshape(-1)
        nidx3 = jnp.concatenate(
            [wh.reshape(_NW, -1), ch.reshape(_NW, -1)], axis=1
        ).reshape(_NW, -1, _CHUNK)
        cidx3 = jnp.concatenate(
            [ch.reshape(_NW, -1), negf.reshape(_NW, -1)], axis=1
        ).reshape(_NW, -1, _CHUNK)
        chunks.append((nidx3, cidx3))

    sq = None
    prior = None
    tot = jnp.float32(0.0)
    for (o, n), (nidx3, cidx3) in zip(zip(offs, sizes), chunks):
        we, ce, cctx, negrows = _sc_gather(node_emb, ctx_emb, nidx3, cidx3,
                                           n, nneg)
        neg3 = negrows.reshape(nneg, n, emb)
        sq, prior, acc = _tc_math(we, ce, cctx, neg3, W_comm, bs, o,
                                  sq, prior)
        tot = tot + acc[0, 0]
    out = -tot / bs
    return (out, sq, prior)
